# baseline (speedup 1.0000x reference)
import functools

import jax
import jax.numpy as jnp
from jax import lax
from jax.experimental import pallas as pl
from jax.experimental.pallas import tpu as pltpu
from jax.experimental.pallas import tpu_sc as plsc

_B = 16
_T = 1024
_N = _B * _T
_IN = 512
_LAT = 256
_TN = 1024
_TR = 1024


def _factor(z, z2, cbn_row, cb, jjf):
    zn = jnp.sum(z * z, axis=1, keepdims=True)
    mm2 = lax.dot_general(z2, cb, (((1,), (1,)), ((), ())),
                          preferred_element_type=jnp.float32)
    d = (zn + cbn_row) - mm2
    dmin = jnp.min(d, axis=1, keepdims=True)
    idxf = jnp.min(jnp.where(d == dmin, jjf, float(d.shape[1])), axis=1)
    return idxf.astype(jnp.int32), jnp.sum(dmin)


def _body1(x_ref, Wall_ref, ball_ref, cbc_ref, cbp_ref, cbt_ref,
           ci_ref, pi_ref, ti_ref, parts_ref, cbn_ref):
    @pl.when(pl.program_id(0) == 0)
    def _init():
        cbn_ref[0:1, :1024] = jnp.sum(cbc_ref[...] * cbc_ref[...],
                                      axis=1).reshape(1, 1024)
        cbn_ref[1:2, :1024] = jnp.sum(cbp_ref[...] * cbp_ref[...],
                                      axis=1).reshape(1, 1024)
        cbn_ref[2:3, :512] = jnp.sum(cbt_ref[...] * cbt_ref[...],
                                     axis=1).reshape(1, 512)

    x = x_ref[...]
    z_all = jnp.dot(x, Wall_ref[...],
                    preferred_element_type=jnp.float32) + ball_ref[...]
    z2_all = z_all + z_all
    jjf = lax.broadcasted_iota(jnp.int32, (_TN, 1024), 1).astype(jnp.float32)
    jjf_t = lax.broadcasted_iota(jnp.int32, (_TN, 512), 1).astype(jnp.float32)
    ic, s_c = _factor(z_all[:, :_LAT], z2_all[:, :_LAT],
                      cbn_ref[0:1, :1024], cbc_ref[...], jjf)
    ip, s_p = _factor(z_all[:, _LAT:2 * _LAT], z2_all[:, _LAT:2 * _LAT],
                      cbn_ref[1:2, :1024], cbp_ref[...], jjf)
    it, s_t = _factor(z_all[:, 2 * _LAT:], z2_all[:, 2 * _LAT:],
                      cbn_ref[2:3, :512], cbt_ref[...], jjf_t)
    ci_ref[...] = ic.reshape(1, 1, _TN)
    pi_ref[...] = ip.reshape(1, 1, _TN)
    ti_ref[...] = it.reshape(1, 1, _TN)
    rr = lax.broadcasted_iota(jnp.int32, (3, 128), 0)
    parts = jnp.where(rr == 0, s_c, jnp.where(rr == 1, s_p, s_t))
    parts_ref[...] = parts.reshape(1, 3, 128)


def _distance_argmin(xf, Wall, ball, cb_c, cb_p, cb_t):
    g = _N // _TN
    full = lambda shape: pl.BlockSpec(shape, lambda i: (0,) * len(shape))
    out_shapes = (
        jax.ShapeDtypeStruct((g, 1, _TN), jnp.int32),
        jax.ShapeDtypeStruct((g, 1, _TN), jnp.int32),
        jax.ShapeDtypeStruct((g, 1, _TN), jnp.int32),
        jax.ShapeDtypeStruct((g, 3, 128), jnp.float32),
    )
    in_specs = [
        pl.BlockSpec((_TN, _IN), lambda i: (i, 0)),
        full((_IN, 3 * _LAT)), full((1, 3 * _LAT)),
        full((1024, _LAT)), full((1024, _LAT)), full((512, _LAT)),
    ]
    out_specs = (
        pl.BlockSpec((1, 1, _TN), lambda i: (i, 0, 0)),
        pl.BlockSpec((1, 1, _TN), lambda i: (i, 0, 0)),
        pl.BlockSpec((1, 1, _TN), lambda i: (i, 0, 0)),
        pl.BlockSpec((1, 3, 128), lambda i: (i, 0, 0)),
    )
    return pl.pallas_call(
        _body1,
        grid=(g,),
        in_specs=in_specs,
        out_specs=out_specs,
        out_shape=out_shapes,
        scratch_shapes=[pltpu.VMEM((3, 1024), jnp.float32)],
        compiler_params=pltpu.CompilerParams(
            dimension_semantics=("arbitrary",)),
    )(xf, Wall, ball, cb_c, cb_p, cb_t)


_NW = 32
_PW = _N // _NW
_CH = 128


def _sc_gather(cb_c, cb_p, cb_t, ci, pi, ti):
    mesh = plsc.VectorSubcoreMesh(core_axis_name="c", subcore_axis_name="s")

    @functools.partial(
        pl.kernel, mesh=mesh,
        out_type=(
            jax.ShapeDtypeStruct((_N, _LAT), jnp.float32),
            jax.ShapeDtypeStruct((_N, _LAT), jnp.float32),
            jax.ShapeDtypeStruct((_N, _LAT), jnp.float32),
        ),
        scratch_types=[
            pltpu.VMEM((_CH,), jnp.int32),
            pltpu.VMEM((_CH, _LAT), jnp.float32),
            pltpu.SemaphoreType.DMA,
        ],
    )
    def k(cbc_hbm, cbp_hbm, cbt_hbm, ci_hbm, pi_hbm, ti_hbm,
          zcq_hbm, zpq_hbm, ztq_hbm, idx_v, rows_v, sem):
        wid = lax.axis_index("s") * 2 + lax.axis_index("c")
        base = wid * _PW
        for tab, idx_hbm, out_hbm in (
                (cbc_hbm, ci_hbm, zcq_hbm),
                (cbp_hbm, pi_hbm, zpq_hbm),
                (cbt_hbm, ti_hbm, ztq_hbm)):
            for c in range(_PW // _CH):
                off = base + c * _CH
                pltpu.sync_copy(idx_hbm.at[pl.ds(off, _CH)], idx_v)
                pltpu.async_copy(tab.at[idx_v], rows_v, sem).wait()
                pltpu.sync_copy(rows_v, out_hbm.at[pl.ds(off, _CH)])

    return k(cb_c, cb_p, cb_t, ci, pi, ti)


def _body3(zcq_ref, zpq_ref, ztq_ref, Wrc_ref, Wrp_ref, Wrt_ref, br_ref,
           xr_ref):
    a = jnp.dot(zcq_ref[...].astype(jnp.bfloat16), Wrc_ref[...],
                preferred_element_type=jnp.float32)
    b = jnp.dot(zpq_ref[...].astype(jnp.bfloat16), Wrp_ref[...],
                preferred_element_type=jnp.float32)
    c = jnp.dot(ztq_ref[...].astype(jnp.bfloat16), Wrt_ref[...],
                preferred_element_type=jnp.float32)
    xr_ref[...] = ((a + b) + c) + br_ref[...]


def _recon(zcq, zpq, ztq, Wr16, br):
    g = _N // _TR
    full = lambda shape: pl.BlockSpec(shape, lambda i: (0,) * len(shape))
    tok = pl.BlockSpec((_TR, _LAT), lambda i: (i, 0))
    return pl.pallas_call(
        _body3,
        grid=(g,),
        in_specs=[tok, tok, tok,
                  full((_LAT, _IN)), full((_LAT, _IN)), full((_LAT, _IN)),
                  full((1, _IN))],
        out_specs=pl.BlockSpec((_TR, _IN), lambda i: (i, 0)),
        out_shape=jax.ShapeDtypeStruct((_N, _IN), jnp.float32),
        compiler_params=pltpu.CompilerParams(
            dimension_semantics=("arbitrary",)),
    )(zcq, zpq, ztq, Wr16[:_LAT], Wr16[_LAT:2 * _LAT], Wr16[2 * _LAT:],
      br.reshape(1, _IN))


@jax.jit
def kernel(x, Wc, bc, Wp, bp, Wt, bt, cb_c, cb_p, cb_t, Wr, br):
    xf = x.reshape(_N, _IN)
    ci, pi, ti, parts = _distance_argmin(
        xf,
        jnp.concatenate([Wc, Wp, Wt], axis=1),
        jnp.concatenate([bc, bp, bt]).reshape(1, 3 * _LAT),
        cb_c, cb_p, cb_t)
    cif, pif, tif = (ci.reshape(_N), pi.reshape(_N), ti.reshape(_N))
    zcq, zpq, ztq = _sc_gather(cb_c, cb_p, cb_t, cif, pif, tif)
    xr = _recon(zcq, zpq, ztq, Wr.astype(jnp.bfloat16), br)
    sums = parts[:, :, 0].sum(axis=0)
    mse_mean = (sums[0] + sums[1] + sums[2]) / (3.0 * _N * _LAT)
    vq_loss = mse_mean + 0.25 * mse_mean
    return (xr.reshape(_B, _T, _IN), vq_loss,
            cif.reshape(_B, _T), pif.reshape(_B, _T), tif.reshape(_B, _T),
            zcq.reshape(_B, _T, _LAT), zpq.reshape(_B, _T, _LAT),
            ztq.reshape(_B, _T, _LAT))


# speedup vs baseline: 1.6423x; 1.6423x over previous
import functools

import jax
import jax.numpy as jnp
from jax import lax
from jax.experimental import pallas as pl
from jax.experimental.pallas import tpu as pltpu

_B = 16
_T = 1024
_N = _B * _T
_IN = 512
_LAT = 256
_TN = 2048


def _factor(z, z2, cbn_row, jjf, cb, cb16):
    zn = jnp.sum(z * z, axis=1, keepdims=True)
    mm2 = lax.dot_general(z2, cb, (((1,), (1,)), ((), ())),
                          preferred_element_type=jnp.float32)
    d = (zn + cbn_row) - mm2
    dmin = jnp.min(d, axis=1, keepdims=True)
    jv = jjf
    idxf = jnp.min(jnp.where(d == dmin, jv, float(d.shape[1])), axis=1)
    idx = idxf.astype(jnp.int32)
    oh = (jv == idxf[:, None]).astype(jnp.bfloat16)
    zq = jnp.dot(oh, cb16, preferred_element_type=jnp.float32)
    return idx, zq, jnp.sum(dmin)


def _body(x_ref, Wall_ref, ball_ref,
          cbc_ref, cbp_ref, cbt_ref, cbc16_ref, cbp16_ref, cbt16_ref,
          Wr16_ref, br_ref,
          xr_ref, ci_ref, pi_ref, ti_ref, zcq_ref, zpq_ref, ztq_ref,
          parts_ref, cbn_ref):
    @pl.when(pl.program_id(0) == 0)
    def _init():
        cbn_ref[0:1, :1024] = jnp.sum(cbc_ref[...] * cbc_ref[...],
                                    axis=1).reshape(1, 1024)
        cbn_ref[1:2, :1024] = jnp.sum(cbp_ref[...] * cbp_ref[...],
                                    axis=1).reshape(1, 1024)
        cbn_ref[2:3, :512] = jnp.sum(cbt_ref[...] * cbt_ref[...],
                                   axis=1).reshape(1, 512)

    x = x_ref[...]
    z_all = jnp.dot(x, Wall_ref[...],
                    preferred_element_type=jnp.float32) + ball_ref[...]
    z2_all = z_all + z_all
    jjf = lax.broadcasted_iota(jnp.int32, (_TN, 1024), 1).astype(jnp.float32)
    jjf_t = lax.broadcasted_iota(jnp.int32, (_TN, 512), 1).astype(jnp.float32)
    ic, zcq, s_c = _factor(z_all[:, :_LAT], z2_all[:, :_LAT],
                           cbn_ref[0:1, :1024], jjf,
                           cbc_ref[...], cbc16_ref[...])
    ip, zpq, s_p = _factor(z_all[:, _LAT:2 * _LAT], z2_all[:, _LAT:2 * _LAT],
                           cbn_ref[1:2, :1024], jjf,
                           cbp_ref[...], cbp16_ref[...])
    it, ztq, s_t = _factor(z_all[:, 2 * _LAT:], z2_all[:, 2 * _LAT:],
                           cbn_ref[2:3, :512], jjf_t,
                           cbt_ref[...], cbt16_ref[...])
    zq = jnp.concatenate([zcq, zpq, ztq], axis=1).astype(jnp.bfloat16)
    xr_ref[...] = (jnp.dot(zq, Wr16_ref[...], preferred_element_type=jnp.float32)
                   + br_ref[...])
    ci_ref[...] = ic.reshape(1, 1, _TN)
    pi_ref[...] = ip.reshape(1, 1, _TN)
    ti_ref[...] = it.reshape(1, 1, _TN)
    zcq_ref[...] = zcq
    zpq_ref[...] = zpq
    ztq_ref[...] = ztq
    rr = lax.broadcasted_iota(jnp.int32, (3, 128), 0)
    parts = jnp.where(rr == 0, s_c, jnp.where(rr == 1, s_p, s_t))
    parts_ref[...] = parts.reshape(1, 3, 128)


@jax.jit
def kernel(x, Wc, bc, Wp, bp, Wt, bt, cb_c, cb_p, cb_t, Wr, br):
    g = _N // _TN
    xf = x.reshape(_N, _IN)
    full = lambda shape: pl.BlockSpec(shape, lambda i: (0,) * len(shape))
    out_shapes = (
        jax.ShapeDtypeStruct((_N, _IN), jnp.float32),
        jax.ShapeDtypeStruct((g, 1, _TN), jnp.int32),
        jax.ShapeDtypeStruct((g, 1, _TN), jnp.int32),
        jax.ShapeDtypeStruct((g, 1, _TN), jnp.int32),
        jax.ShapeDtypeStruct((_N, _LAT), jnp.float32),
        jax.ShapeDtypeStruct((_N, _LAT), jnp.float32),
        jax.ShapeDtypeStruct((_N, _LAT), jnp.float32),
        jax.ShapeDtypeStruct((g, 3, 128), jnp.float32),
    )
    in_specs = [
        pl.BlockSpec((_TN, _IN), lambda i: (i, 0)),
        full((_IN, 3 * _LAT)), full((1, 3 * _LAT)),
        full((1024, _LAT)), full((1024, _LAT)), full((512, _LAT)),
        full((1024, _LAT)), full((1024, _LAT)), full((512, _LAT)),
        full((3 * _LAT, _IN)), full((1, _IN)),
    ]
    out_specs = (
        pl.BlockSpec((_TN, _IN), lambda i: (i, 0)),
        pl.BlockSpec((1, 1, _TN), lambda i: (i, 0, 0)),
        pl.BlockSpec((1, 1, _TN), lambda i: (i, 0, 0)),
        pl.BlockSpec((1, 1, _TN), lambda i: (i, 0, 0)),
        pl.BlockSpec((_TN, _LAT), lambda i: (i, 0)),
        pl.BlockSpec((_TN, _LAT), lambda i: (i, 0)),
        pl.BlockSpec((_TN, _LAT), lambda i: (i, 0)),
        pl.BlockSpec((1, 3, 128), lambda i: (i, 0, 0)),
    )
    outs = pl.pallas_call(
        _body,
        grid=(g,),
        in_specs=in_specs,
        out_specs=out_specs,
        out_shape=out_shapes,
        scratch_shapes=[pltpu.VMEM((3, 1024), jnp.float32)],
        compiler_params=pltpu.CompilerParams(
            dimension_semantics=("arbitrary",)),
    )(xf,
      jnp.concatenate([Wc, Wp, Wt], axis=1),
      jnp.concatenate([bc, bp, bt]).reshape(1, 3 * _LAT),
      cb_c, cb_p, cb_t,
      cb_c.astype(jnp.bfloat16), cb_p.astype(jnp.bfloat16),
      cb_t.astype(jnp.bfloat16),
      Wr.astype(jnp.bfloat16), br.reshape(1, _IN))
    xr, ci, pi, ti, zcq, zpq, ztq, parts = outs
    sums = parts[:, :, 0].sum(axis=0)
    mse_mean = (sums[0] + sums[1] + sums[2]) / (3.0 * _N * _LAT)
    vq_loss = mse_mean + 0.25 * mse_mean
    return (xr.reshape(_B, _T, _IN), vq_loss,
            ci.reshape(_B, _T), pi.reshape(_B, _T), ti.reshape(_B, _T),
            zcq.reshape(_B, _T, _LAT), zpq.reshape(_B, _T, _LAT),
            ztq.reshape(_B, _T, _LAT))


# speedup vs baseline: 1.6566x; 1.0088x over previous
import functools

import jax
import jax.numpy as jnp
from jax import lax
from jax.experimental import pallas as pl
from jax.experimental.pallas import tpu as pltpu

_B = 16
_T = 1024
_N = _B * _T
_IN = 512
_LAT = 256
_TN = 1024


def _factor(z, z2, cbn_row, jjf, cb, cb16):
    zn = jnp.sum(z * z, axis=1, keepdims=True)
    mm2 = lax.dot_general(z2, cb, (((1,), (1,)), ((), ())),
                          preferred_element_type=jnp.float32)
    d = (zn + cbn_row) - mm2
    dmin = jnp.min(d, axis=1, keepdims=True)
    idxf = jnp.min(jnp.where(d == dmin, jjf, float(d.shape[1])), axis=1)
    idx = idxf.astype(jnp.int32)
    oh = (jjf == idxf[:, None]).astype(jnp.bfloat16)
    zq = jnp.dot(oh, cb16, preferred_element_type=jnp.float32)
    return idx, zq, jnp.sum(dmin)


def _body(x_ref, Wall_ref, ball_ref,
          cbc_ref, cbp_ref, cbt_ref, cbc16_ref, cbp16_ref, cbt16_ref,
          Wr16_ref, br_ref,
          xr_ref, ci_ref, pi_ref, ti_ref, zcq_ref, zpq_ref, ztq_ref,
          parts_ref, cbn_ref):
    @pl.when(pl.program_id(0) == 0)
    def _init():
        cbn_ref[0:1, :1024] = jnp.sum(cbc_ref[...] * cbc_ref[...],
                                    axis=1).reshape(1, 1024)
        cbn_ref[1:2, :1024] = jnp.sum(cbp_ref[...] * cbp_ref[...],
                                    axis=1).reshape(1, 1024)
        cbn_ref[2:3, :512] = jnp.sum(cbt_ref[...] * cbt_ref[...],
                                   axis=1).reshape(1, 512)

    x = x_ref[...]
    z_all = jnp.dot(x, Wall_ref[...],
                    preferred_element_type=jnp.float32) + ball_ref[...]
    z2_all = z_all + z_all
    jjf = lax.broadcasted_iota(jnp.int32, (_TN, 1024), 1).astype(jnp.float32)
    jjf_t = lax.broadcasted_iota(jnp.int32, (_TN, 512), 1).astype(jnp.float32)
    ic, zcq, s_c = _factor(z_all[:, :_LAT], z2_all[:, :_LAT],
                           cbn_ref[0:1, :1024], jjf,
                           cbc_ref[...], cbc16_ref[...])
    ip, zpq, s_p = _factor(z_all[:, _LAT:2 * _LAT], z2_all[:, _LAT:2 * _LAT],
                           cbn_ref[1:2, :1024], jjf,
                           cbp_ref[...], cbp16_ref[...])
    it, ztq, s_t = _factor(z_all[:, 2 * _LAT:], z2_all[:, 2 * _LAT:],
                           cbn_ref[2:3, :512], jjf_t,
                           cbt_ref[...], cbt16_ref[...])
    zq = jnp.concatenate([zcq, zpq, ztq], axis=1).astype(jnp.bfloat16)
    xr_ref[...] = (jnp.dot(zq, Wr16_ref[...], preferred_element_type=jnp.float32)
                   + br_ref[...])
    ci_ref[...] = ic.reshape(1, 1, _TN)
    pi_ref[...] = ip.reshape(1, 1, _TN)
    ti_ref[...] = it.reshape(1, 1, _TN)
    zcq_ref[...] = zcq
    zpq_ref[...] = zpq
    ztq_ref[...] = ztq
    rr = lax.broadcasted_iota(jnp.int32, (3, 128), 0)
    parts = jnp.where(rr == 0, s_c, jnp.where(rr == 1, s_p, s_t))
    parts_ref[...] = parts.reshape(1, 3, 128)


@jax.jit
def kernel(x, Wc, bc, Wp, bp, Wt, bt, cb_c, cb_p, cb_t, Wr, br):
    g = _N // _TN
    xf = x.reshape(_N, _IN)
    full = lambda shape: pl.BlockSpec(shape, lambda i: (0,) * len(shape))
    out_shapes = (
        jax.ShapeDtypeStruct((_N, _IN), jnp.float32),
        jax.ShapeDtypeStruct((g, 1, _TN), jnp.int32),
        jax.ShapeDtypeStruct((g, 1, _TN), jnp.int32),
        jax.ShapeDtypeStruct((g, 1, _TN), jnp.int32),
        jax.ShapeDtypeStruct((_N, _LAT), jnp.float32),
        jax.ShapeDtypeStruct((_N, _LAT), jnp.float32),
        jax.ShapeDtypeStruct((_N, _LAT), jnp.float32),
        jax.ShapeDtypeStruct((g, 3, 128), jnp.float32),
    )
    in_specs = [
        pl.BlockSpec((_TN, _IN), lambda i: (i, 0)),
        full((_IN, 3 * _LAT)), full((1, 3 * _LAT)),
        full((1024, _LAT)), full((1024, _LAT)), full((512, _LAT)),
        full((1024, _LAT)), full((1024, _LAT)), full((512, _LAT)),
        full((3 * _LAT, _IN)), full((1, _IN)),
    ]
    out_specs = (
        pl.BlockSpec((_TN, _IN), lambda i: (i, 0)),
        pl.BlockSpec((1, 1, _TN), lambda i: (i, 0, 0)),
        pl.BlockSpec((1, 1, _TN), lambda i: (i, 0, 0)),
        pl.BlockSpec((1, 1, _TN), lambda i: (i, 0, 0)),
        pl.BlockSpec((_TN, _LAT), lambda i: (i, 0)),
        pl.BlockSpec((_TN, _LAT), lambda i: (i, 0)),
        pl.BlockSpec((_TN, _LAT), lambda i: (i, 0)),
        pl.BlockSpec((1, 3, 128), lambda i: (i, 0, 0)),
    )
    outs = pl.pallas_call(
        _body,
        grid=(g,),
        in_specs=in_specs,
        out_specs=out_specs,
        out_shape=out_shapes,
        scratch_shapes=[pltpu.VMEM((3, 1024), jnp.float32)],
        compiler_params=pltpu.CompilerParams(
            dimension_semantics=("arbitrary",)),
    )(xf,
      jnp.concatenate([Wc, Wp, Wt], axis=1),
      jnp.concatenate([bc, bp, bt]).reshape(1, 3 * _LAT),
      cb_c, cb_p, cb_t,
      cb_c.astype(jnp.bfloat16), cb_p.astype(jnp.bfloat16),
      cb_t.astype(jnp.bfloat16),
      Wr.astype(jnp.bfloat16), br.reshape(1, _IN))
    xr, ci, pi, ti, zcq, zpq, ztq, parts = outs
    sums = parts[:, :, 0].sum(axis=0)
    mse_mean = (sums[0] + sums[1] + sums[2]) / (3.0 * _N * _LAT)
    vq_loss = mse_mean + 0.25 * mse_mean
    return (xr.reshape(_B, _T, _IN), vq_loss,
            ci.reshape(_B, _T), pi.reshape(_B, _T), ti.reshape(_B, _T),
            zcq.reshape(_B, _T, _LAT), zpq.reshape(_B, _T, _LAT),
            ztq.reshape(_B, _T, _LAT))


# speedup vs baseline: 1.6630x; 1.0038x over previous
import jax
import jax.numpy as jnp
from jax import lax
from jax.experimental import pallas as pl
from jax.experimental.pallas import tpu as pltpu

_B = 16
_T = 1024
_N = _B * _T
_IN = 512
_LAT = 256
_TN = 1024


def _factor(z, z2, cbn_row, jjf, cb, cb16):
    zn = jnp.sum(z * z, axis=1, keepdims=True)
    mm2 = lax.dot_general(z2, cb, (((1,), (1,)), ((), ())),
                          preferred_element_type=jnp.float32)
    d = (zn + cbn_row) - mm2
    dmin = jnp.min(d, axis=1, keepdims=True)
    idxf = jnp.min(jnp.where(d == dmin, jjf, float(d.shape[1])), axis=1)
    idx = idxf.astype(jnp.int32)
    oh = (jjf == idxf[:, None]).astype(jnp.bfloat16)
    zq = jnp.dot(oh, cb16, preferred_element_type=jnp.float32)
    return idx, zq, jnp.sum(dmin)


def _body(x_ref, Wall_ref, ball_ref,
          cbc_ref, cbp_ref, cbt_ref, cbc16_ref, cbp16_ref, cbt16_ref,
          Wr16_ref, br_ref,
          xr_ref, ci_ref, pi_ref, ti_ref, zcq_ref, zpq_ref, ztq_ref,
          parts_ref, cbn_ref):
    @pl.when(pl.program_id(0) == 0)
    def _init():
        cbn_ref[0:1, :1024] = jnp.sum(cbc_ref[...] * cbc_ref[...],
                                    axis=1).reshape(1, 1024)
        cbn_ref[1:2, :1024] = jnp.sum(cbp_ref[...] * cbp_ref[...],
                                    axis=1).reshape(1, 1024)
        cbn_ref[2:3, :512] = jnp.sum(cbt_ref[...] * cbt_ref[...],
                                   axis=1).reshape(1, 512)

    x = x_ref[...]
    z_all = jnp.dot(x, Wall_ref[...],
                    preferred_element_type=jnp.float32) + ball_ref[...]
    z2_all = z_all + z_all
    jjf = lax.broadcasted_iota(jnp.int32, (_TN, 1024), 1).astype(jnp.float32)
    jjf_t = lax.broadcasted_iota(jnp.int32, (_TN, 512), 1).astype(jnp.float32)
    ic, zcq, s_c = _factor(z_all[:, :_LAT], z2_all[:, :_LAT],
                           cbn_ref[0:1, :1024], jjf,
                           cbc_ref[...], cbc16_ref[...])
    ip, zpq, s_p = _factor(z_all[:, _LAT:2 * _LAT], z2_all[:, _LAT:2 * _LAT],
                           cbn_ref[1:2, :1024], jjf,
                           cbp_ref[...], cbp16_ref[...])
    it, ztq, s_t = _factor(z_all[:, 2 * _LAT:], z2_all[:, 2 * _LAT:],
                           cbn_ref[2:3, :512], jjf_t,
                           cbt_ref[...], cbt16_ref[...])
    zq = jnp.concatenate([zcq, zpq, ztq], axis=1).astype(jnp.bfloat16)
    xr_ref[...] = (jnp.dot(zq, Wr16_ref[...], preferred_element_type=jnp.float32)
                   + br_ref[...])
    ci_ref[...] = ic.reshape(1, 1, _TN)
    pi_ref[...] = ip.reshape(1, 1, _TN)
    ti_ref[...] = it.reshape(1, 1, _TN)
    zcq_ref[...] = zcq
    zpq_ref[...] = zpq
    ztq_ref[...] = ztq
    rr = lax.broadcasted_iota(jnp.int32, (3, 128), 0)
    parts = jnp.where(rr == 0, s_c, jnp.where(rr == 1, s_p, s_t))
    parts_ref[...] = parts.reshape(1, 3, 128)


@jax.jit
def kernel(x, Wc, bc, Wp, bp, Wt, bt, cb_c, cb_p, cb_t, Wr, br):
    g = _N // _TN
    xf = x.reshape(_N, _IN)
    full = lambda shape: pl.BlockSpec(shape, lambda i: (0,) * len(shape))
    out_shapes = (
        jax.ShapeDtypeStruct((_N, _IN), jnp.float32),
        jax.ShapeDtypeStruct((g, 1, _TN), jnp.int32),
        jax.ShapeDtypeStruct((g, 1, _TN), jnp.int32),
        jax.ShapeDtypeStruct((g, 1, _TN), jnp.int32),
        jax.ShapeDtypeStruct((_N, _LAT), jnp.float32),
        jax.ShapeDtypeStruct((_N, _LAT), jnp.float32),
        jax.ShapeDtypeStruct((_N, _LAT), jnp.float32),
        jax.ShapeDtypeStruct((g, 3, 128), jnp.float32),
    )
    in_specs = [
        pl.BlockSpec((_TN, _IN), lambda i: (i, 0)),
        full((_IN, 3 * _LAT)), full((1, 3 * _LAT)),
        full((1024, _LAT)), full((1024, _LAT)), full((512, _LAT)),
        full((1024, _LAT)), full((1024, _LAT)), full((512, _LAT)),
        full((3 * _LAT, _IN)), full((1, _IN)),
    ]
    out_specs = (
        pl.BlockSpec((_TN, _IN), lambda i: (i, 0)),
        pl.BlockSpec((1, 1, _TN), lambda i: (i, 0, 0)),
        pl.BlockSpec((1, 1, _TN), lambda i: (i, 0, 0)),
        pl.BlockSpec((1, 1, _TN), lambda i: (i, 0, 0)),
        pl.BlockSpec((_TN, _LAT), lambda i: (i, 0)),
        pl.BlockSpec((_TN, _LAT), lambda i: (i, 0)),
        pl.BlockSpec((_TN, _LAT), lambda i: (i, 0)),
        pl.BlockSpec((1, 3, 128), lambda i: (i, 0, 0)),
    )
    outs = pl.pallas_call(
        _body,
        grid=(g,),
        in_specs=in_specs,
        out_specs=out_specs,
        out_shape=out_shapes,
        scratch_shapes=[pltpu.VMEM((3, 1024), jnp.float32)],
        compiler_params=pltpu.CompilerParams(
            dimension_semantics=("arbitrary",)),
    )(xf,
      jnp.concatenate([Wc, Wp, Wt], axis=1),
      jnp.concatenate([bc, bp, bt]).reshape(1, 3 * _LAT),
      cb_c, cb_p, cb_t,
      cb_c.astype(jnp.bfloat16), cb_p.astype(jnp.bfloat16),
      cb_t.astype(jnp.bfloat16),
      Wr.astype(jnp.bfloat16), br.reshape(1, _IN))
    xr, ci, pi, ti, zcq, zpq, ztq, parts = outs
    sums = parts[:, :, 0].sum(axis=0)
    mse_mean = (sums[0] + sums[1] + sums[2]) / (3.0 * _N * _LAT)
    vq_loss = mse_mean + 0.25 * mse_mean
    return (xr.reshape(_B, _T, _IN), vq_loss,
            ci.reshape(_B, _T), pi.reshape(_B, _T), ti.reshape(_B, _T),
            zcq.reshape(_B, _T, _LAT), zpq.reshape(_B, _T, _LAT),
            ztq.reshape(_B, _T, _LAT))
